# baseline (speedup 1.0000x reference)
import functools

import jax
import jax.numpy as jnp
from jax.experimental import pallas as pl
from jax.experimental.pallas import tpu as pltpu

N, D, H, G = 10000, 128, 128, 16
BN = 5000
NSTEPS = N // BN


def _fused_kernel(x_ref, batch_ref, Wg1_ref, bg1_ref, Wg2_ref, bg2_ref,
                  Wf1_ref, bf1_ref, Wf2_ref, bf2_ref, out_ref,
                  sums_ref, counts_ref):
    i = pl.program_id(0)

    Wg1 = Wg1_ref[...]
    Wg2 = Wg2_ref[...]
    bg1 = bg1_ref[...]
    bg2 = bg2_ref[...]
    b = batch_ref[0, 0, :]

    SB = 1000
    part_sums = jnp.zeros((G, H), jnp.float32)
    part_counts = jnp.zeros((G, 1), jnp.float32)
    seg = jax.lax.broadcasted_iota(jnp.int32, (G, SB), 0)
    for sb in range(BN // SB):
        x = x_ref[sb * SB:(sb + 1) * SB, :]
        y1 = jnp.dot(x, Wg1, precision=jax.lax.Precision.DEFAULT,
                     preferred_element_type=jnp.float32)
        h = jnp.maximum(y1 + bg1, 0.0)
        y2 = jnp.dot(h, Wg2, precision=jax.lax.Precision.DEFAULT,
                     preferred_element_type=jnp.float32)
        h = jnp.maximum(y2 + bg2, 0.0)
        onehot_f = (b[None, sb * SB:(sb + 1) * SB] == seg).astype(jnp.float32)
        part_sums += jnp.dot(onehot_f, h, precision=jax.lax.Precision.DEFAULT,
                             preferred_element_type=jnp.float32)
        part_counts += jnp.sum(onehot_f, axis=1, keepdims=True)

    @pl.when(i == 0)
    def _init():
        sums_ref[...] = part_sums
        counts_ref[...] = part_counts

    @pl.when(i > 0)
    def _acc():
        sums_ref[...] += part_sums
        counts_ref[...] += part_counts

    @pl.when(i == NSTEPS - 1)
    def _finalize():
        pooled = sums_ref[...] / jnp.maximum(counts_ref[...], 1.0)
        h2 = jnp.maximum(jnp.dot(pooled, Wf1_ref[...],
                                 preferred_element_type=jnp.float32)
                         + bf1_ref[...], 0.0)
        out_ref[...] = jnp.dot(h2, Wf2_ref[...],
                               preferred_element_type=jnp.float32) + bf2_ref[...]


@jax.jit
def _run(x, batch, Wg1, bg1, Wg2, bg2, Wf1, bf1, Wf2, bf2):
    batch3 = batch.reshape(NSTEPS, 1, BN)
    full = lambda shape: pl.BlockSpec(shape, lambda i: (0,) * len(shape))
    return pl.pallas_call(
        _fused_kernel,
        grid=(NSTEPS,),
        in_specs=[
            pl.BlockSpec((BN, D), lambda i: (i, 0)),
            pl.BlockSpec((1, 1, BN), lambda i: (i, 0, 0)),
            full((D, H)), full((H,)), full((H, H)), full((H,)),
            full((H, H)), full((H,)), full((H, H)), full((H,)),
        ],
        out_specs=pl.BlockSpec((G, H), lambda i: (0, 0)),
        out_shape=jax.ShapeDtypeStruct((G, H), jnp.float32),
        scratch_shapes=[
            pltpu.VMEM((G, H), jnp.float32),
            pltpu.VMEM((G, 1), jnp.float32),
        ],
    )(x, batch3, Wg1, bg1, Wg2, bg2, Wf1, bf1, Wf2, bf2)


def kernel(x, edge_index, batch, Wg1, bg1, Wg2, bg2, Wf1, bf1, Wf2, bf2):
    del edge_index
    return _run(x, batch, Wg1, bg1, Wg2, bg2, Wf1, bf1, Wf2, bf2)


# speedup vs baseline: 1.1033x; 1.1033x over previous
import jax
import jax.numpy as jnp
from jax.experimental import pallas as pl
from jax.experimental.pallas import tpu as pltpu

N, D, H, G = 10000, 128, 128, 16
HB = N // 2


def _fused_kernel(xt_ref, xb_ref, batch_ref, Wg1_ref, bg1_ref, Wg2_ref,
                  bg2_ref, Wf1_ref, bf1_ref, Wf2_ref, bf2_ref, out_ref):
    Wg1 = Wg1_ref[...]
    Wg2 = Wg2_ref[...]
    bg1 = bg1_ref[...]
    bg2 = bg2_ref[...]

    sums = jnp.zeros((G, H), jnp.float32)
    counts = jnp.zeros((G, 1), jnp.float32)
    seg = jax.lax.broadcasted_iota(jnp.int32, (G, HB), 0)
    for half, x_ref in enumerate((xt_ref, xb_ref)):
        x = x_ref[...]
        y1 = jnp.dot(x, Wg1, precision=jax.lax.Precision.DEFAULT,
                     preferred_element_type=jnp.float32)
        h = jnp.maximum(y1 + bg1, 0.0)
        y2 = jnp.dot(h, Wg2, precision=jax.lax.Precision.DEFAULT,
                     preferred_element_type=jnp.float32)
        h = jnp.maximum(y2 + bg2, 0.0)
        b = batch_ref[half, 0, :]
        onehot_f = (b[None, :] == seg).astype(jnp.float32)
        sums += jnp.dot(onehot_f, h, precision=jax.lax.Precision.DEFAULT,
                        preferred_element_type=jnp.float32)
        counts += jnp.sum(onehot_f, axis=1, keepdims=True)

    pooled = sums / jnp.maximum(counts, 1.0)
    h2 = jnp.maximum(jnp.dot(pooled, Wf1_ref[...],
                             preferred_element_type=jnp.float32)
                     + bf1_ref[...], 0.0)
    out_ref[...] = jnp.dot(h2, Wf2_ref[...],
                           preferred_element_type=jnp.float32) + bf2_ref[...]


@jax.jit
def _run(x, batch, Wg1, bg1, Wg2, bg2, Wf1, bf1, Wf2, bf2):
    batch3 = batch.reshape(2, 1, HB)
    full = lambda shape: pl.BlockSpec(shape, lambda i: (0,) * len(shape))
    return pl.pallas_call(
        _fused_kernel,
        grid=(1,),
        in_specs=[
            pl.BlockSpec((HB, D), lambda i: (0, 0)),
            pl.BlockSpec((HB, D), lambda i: (1, 0)),
            full((2, 1, HB)),
            full((D, H)), full((H,)), full((H, H)), full((H,)),
            full((H, H)), full((H,)), full((H, H)), full((H,)),
        ],
        out_specs=pl.BlockSpec((G, H), lambda i: (0, 0)),
        out_shape=jax.ShapeDtypeStruct((G, H), jnp.float32),
    )(x, x, batch3, Wg1, bg1, Wg2, bg2, Wf1, bf1, Wf2, bf2)


def kernel(x, edge_index, batch, Wg1, bg1, Wg2, bg2, Wf1, bf1, Wf2, bf2):
    del edge_index
    return _run(x, batch, Wg1, bg1, Wg2, bg2, Wf1, bf1, Wf2, bf2)


# speedup vs baseline: 1.1475x; 1.0401x over previous
import functools

import jax
import jax.numpy as jnp
from jax.experimental import pallas as pl
from jax.experimental.pallas import tpu as pltpu

N, D, H, G = 10000, 128, 128, 16
BN = 5000
NSTEPS = N // BN


def _fused_kernel(x_ref, batch_ref, Wg1_ref, bg1_ref, Wg2_ref, bg2_ref,
                  Wf1_ref, bf1_ref, Wf2_ref, bf2_ref, out_ref,
                  sums_ref, counts_ref):
    i = pl.program_id(0)

    x = x_ref[...]
    y1 = jnp.dot(x, Wg1_ref[...], precision=jax.lax.Precision.DEFAULT,
                 preferred_element_type=jnp.float32)
    h = jnp.maximum(y1 + bg1_ref[...], 0.0)
    y2 = jnp.dot(h, Wg2_ref[...], precision=jax.lax.Precision.DEFAULT,
                 preferred_element_type=jnp.float32)
    h = jnp.maximum(y2 + bg2_ref[...], 0.0)

    b = batch_ref[0, 0, :]
    seg = jax.lax.broadcasted_iota(jnp.int32, (G, BN), 0)
    onehot_f = (b[None, :] == seg).astype(jnp.float32)
    part_sums = jnp.dot(onehot_f, h, precision=jax.lax.Precision.DEFAULT,
                        preferred_element_type=jnp.float32)
    part_counts = jnp.sum(onehot_f, axis=1, keepdims=True)

    @pl.when(i == 0)
    def _init():
        sums_ref[...] = part_sums
        counts_ref[...] = part_counts

    @pl.when(i > 0)
    def _acc():
        sums_ref[...] += part_sums
        counts_ref[...] += part_counts

    @pl.when(i == NSTEPS - 1)
    def _finalize():
        pooled = sums_ref[...] / jnp.maximum(counts_ref[...], 1.0)
        h2 = jnp.maximum(jnp.dot(pooled, Wf1_ref[...],
                                 preferred_element_type=jnp.float32)
                         + bf1_ref[...], 0.0)
        out_ref[...] = jnp.dot(h2, Wf2_ref[...],
                               preferred_element_type=jnp.float32) + bf2_ref[...]


@jax.jit
def _run(x, batch, Wg1, bg1, Wg2, bg2, Wf1, bf1, Wf2, bf2):
    batch3 = batch.reshape(NSTEPS, 1, BN)
    full = lambda shape: pl.BlockSpec(shape, lambda i: (0,) * len(shape))
    return pl.pallas_call(
        _fused_kernel,
        grid=(NSTEPS,),
        in_specs=[
            pl.BlockSpec((BN, D), lambda i: (i, 0)),
            pl.BlockSpec((1, 1, BN), lambda i: (i, 0, 0)),
            full((D, H)), full((H,)), full((H, H)), full((H,)),
            full((H, H)), full((H,)), full((H, H)), full((H,)),
        ],
        out_specs=pl.BlockSpec((G, H), lambda i: (0, 0)),
        out_shape=jax.ShapeDtypeStruct((G, H), jnp.float32),
        scratch_shapes=[
            pltpu.VMEM((G, H), jnp.float32),
            pltpu.VMEM((G, 1), jnp.float32),
        ],
    )(x, batch3, Wg1, bg1, Wg2, bg2, Wf1, bf1, Wf2, bf2)


def kernel(x, edge_index, batch, Wg1, bg1, Wg2, bg2, Wf1, bf1, Wf2, bf2):
    del edge_index
    return _run(x, batch, Wg1, bg1, Wg2, bg2, Wf1, bf1, Wf2, bf2)
